# baseline (speedup 1.0000x reference)
import functools

import jax
import jax.numpy as jnp
from jax import lax
from jax.experimental import pallas as pl
from jax.experimental.pallas import tpu as pltpu
from jax.experimental.pallas import tpu_sc as plsc

VOCAB = 100000
EMB_DIM = 16
OUT_DIM = 10
BATCH = 16384
EPS = 1e-5


def _make_sc_gather(batch, dim):
    info = plsc.get_sparse_core_info()
    nc, ns = info.num_cores, info.num_subcores
    nw = nc * ns
    assert batch % (8 * nw) == 0
    b_per_w = batch // nw
    mesh = plsc.VectorSubcoreMesh(core_axis_name="c", subcore_axis_name="s")

    @functools.partial(
        pl.kernel,
        mesh=mesh,
        out_type=jax.ShapeDtypeStruct((batch, dim), jnp.float32),
        scratch_types=[
            pltpu.VMEM((b_per_w,), jnp.int32),
            pltpu.VMEM((b_per_w, dim), jnp.float32),
            pltpu.SemaphoreType.DMA,
        ],
        compiler_params=pltpu.CompilerParams(use_tc_tiling_on_sc=False),
    )
    def gather_kernel(table_hbm, idx_hbm, out_hbm, idx_v, rows_v, sem):
        wid = lax.axis_index("s") * nc + lax.axis_index("c")
        base = wid * b_per_w
        pltpu.sync_copy(idx_hbm.at[pl.ds(base, b_per_w)], idx_v)
        pltpu.async_copy(table_hbm.at[idx_v], rows_v, sem).wait()
        pltpu.sync_copy(rows_v, out_hbm.at[pl.ds(base, b_per_w)])

    return gather_kernel


def _tc_tail_body(ages_ref, feats_ref, gamma_ref, beta_ref, w_ref, b_ref,
                  out_ref):
    a = ages_ref[...]
    n = a.shape[0]
    mean = jnp.sum(a) / n
    centered = a - mean
    var = jnp.sum(centered * centered) / n
    a_hat = centered * lax.rsqrt(var + EPS)
    age_feat = a_hat * gamma_ref[0] + beta_ref[0]
    age_relu = jnp.maximum(age_feat, 0.0)
    feats = jnp.maximum(feats_ref[...], 0.0)
    w = w_ref[...]
    out = lax.dot_general(
        feats, w[:EMB_DIM, :],
        (((1,), (0,)), ((), ())),
        preferred_element_type=jnp.float32,
    )
    out = out + age_relu * w[EMB_DIM:EMB_DIM + 1, :]
    out = out + b_ref[...]
    out_ref[...] = out


def kernel(customer_ids, ages, emb_table, bn_gamma, bn_beta, W, b):
    feats = _make_sc_gather(BATCH, EMB_DIM)(
        emb_table, customer_ids.astype(jnp.int32))
    out = pl.pallas_call(
        _tc_tail_body,
        out_shape=jax.ShapeDtypeStruct((BATCH, OUT_DIM), jnp.float32),
        in_specs=[
            pl.BlockSpec(memory_space=pltpu.VMEM),
            pl.BlockSpec(memory_space=pltpu.VMEM),
            pl.BlockSpec(memory_space=pltpu.SMEM),
            pl.BlockSpec(memory_space=pltpu.SMEM),
            pl.BlockSpec(memory_space=pltpu.VMEM),
            pl.BlockSpec(memory_space=pltpu.VMEM),
        ],
        out_specs=pl.BlockSpec(memory_space=pltpu.VMEM),
    )(ages.reshape(BATCH, 1), feats, bn_gamma, bn_beta, W,
      b.reshape(1, OUT_DIM))
    return out


# speedup vs baseline: 1.7099x; 1.7099x over previous
import functools

import jax
import jax.numpy as jnp
from jax import lax
from jax.experimental import pallas as pl
from jax.experimental.pallas import tpu as pltpu
from jax.experimental.pallas import tpu_sc as plsc

VOCAB = 100000
EMB_DIM = 16
OUT_DIM = 10
BATCH = 16384
EPS = 1e-5


def _make_sc_gather(batch, dim, vocab):
    info = plsc.get_sparse_core_info()
    nc, ns = info.num_cores, info.num_subcores
    nw = nc * ns
    assert batch % (16 * nw) == 0
    bw = batch // nw
    mesh = plsc.VectorSubcoreMesh(core_axis_name="c", subcore_axis_name="s")

    @functools.partial(
        pl.kernel,
        mesh=mesh,
        out_type=jax.ShapeDtypeStruct((dim, batch), jnp.float32),
        scratch_types=[
            pltpu.VMEM_SHARED((dim, vocab), jnp.float32),
            pltpu.VMEM((bw,), jnp.int32),
            pltpu.VMEM((dim, bw), jnp.float32),
            pltpu.SemaphoreType.DMA,
            pltpu.SemaphoreType.DMA,
        ],
        compiler_params=pltpu.CompilerParams(use_tc_tiling_on_sc=False),
    )
    def gather_kernel(table_hbm, idx_hbm, out_hbm, shared_t, idx_v, stg,
                      sem_t, sem):
        sid = lax.axis_index("s")
        wid = sid * nc + lax.axis_index("c")
        base = wid * bw
        pltpu.async_copy(table_hbm.at[pl.ds(sid, 1), :],
                         shared_t.at[pl.ds(sid, 1), :], sem_t)
        pltpu.sync_copy(idx_hbm.at[pl.ds(base, bw)], idx_v)
        pltpu.make_async_copy(table_hbm.at[pl.ds(sid, 1), :],
                              shared_t.at[pl.ds(sid, 1), :], sem_t).wait()
        plsc.subcore_barrier()

        def chunk(g, carry):
            v = idx_v[pl.ds(g * 16, 16)]
            copies = []
            for j in range(16):
                copies.append(pltpu.async_copy(
                    shared_t.at[:, pl.ds(v[j], 1)],
                    stg.at[:, pl.ds(g * 16 + j, 1)], sem))
            for cp in copies:
                cp.wait()
            return carry

        lax.fori_loop(0, bw // 16, chunk, 0)
        pltpu.sync_copy(stg, out_hbm.at[:, pl.ds(base, bw)])

    return gather_kernel


def _tc_tail_body(ages_ref, feats_ref, gamma_ref, beta_ref, w_ref, b_ref,
                  out_ref):
    a = ages_ref[...]
    n = a.shape[1]
    mean = jnp.sum(a) / n
    centered = a - mean
    var = jnp.sum(centered * centered) / n
    a_hat = centered * lax.rsqrt(var + EPS)
    age_feat = a_hat * gamma_ref[0] + beta_ref[0]
    age_relu = jnp.maximum(age_feat, 0.0)
    feats = jnp.maximum(feats_ref[...], 0.0)
    w = w_ref[...]
    out = lax.dot_general(
        w[:EMB_DIM, :], feats,
        (((0,), (0,)), ((), ())),
        preferred_element_type=jnp.float32,
    )
    out = out + w[EMB_DIM:EMB_DIM + 1, :].reshape(OUT_DIM, 1) * age_relu
    out = out + b_ref[...].reshape(OUT_DIM, 1)
    out_ref[...] = out


def kernel(customer_ids, ages, emb_table, bn_gamma, bn_beta, W, b):
    table_t = emb_table.T
    feats_t = _make_sc_gather(BATCH, EMB_DIM, VOCAB)(
        table_t, customer_ids.astype(jnp.int32))
    out_t = pl.pallas_call(
        _tc_tail_body,
        out_shape=jax.ShapeDtypeStruct((OUT_DIM, BATCH), jnp.float32),
        in_specs=[
            pl.BlockSpec(memory_space=pltpu.VMEM),
            pl.BlockSpec(memory_space=pltpu.VMEM),
            pl.BlockSpec(memory_space=pltpu.SMEM),
            pl.BlockSpec(memory_space=pltpu.SMEM),
            pl.BlockSpec(memory_space=pltpu.VMEM),
            pl.BlockSpec(memory_space=pltpu.VMEM),
        ],
        out_specs=pl.BlockSpec(memory_space=pltpu.VMEM),
    )(ages.reshape(1, BATCH), feats_t, bn_gamma, bn_beta, W,
      b.reshape(1, OUT_DIM))
    return out_t.T


# speedup vs baseline: 1.7825x; 1.0425x over previous
import functools

import jax
import jax.numpy as jnp
from jax import lax
from jax.experimental import pallas as pl
from jax.experimental.pallas import tpu as pltpu
from jax.experimental.pallas import tpu_sc as plsc

VOCAB = 100000
EMB_DIM = 16
OUT_DIM = 10
BATCH = 16384
EPS = 1e-5


def _make_sc_gather(batch, dim, vocab):
    info = plsc.get_sparse_core_info()
    nc, ns = info.num_cores, info.num_subcores
    nw = nc * ns
    assert batch % (16 * nw) == 0
    bw = batch // nw
    mesh = plsc.VectorSubcoreMesh(core_axis_name="c", subcore_axis_name="s")

    @functools.partial(
        pl.kernel,
        mesh=mesh,
        out_type=jax.ShapeDtypeStruct((dim, batch), jnp.float32),
        scratch_types=[
            pltpu.VMEM_SHARED((dim, vocab), jnp.float32),
            pltpu.VMEM((bw,), jnp.int32),
            pltpu.VMEM((dim, bw), jnp.float32),
            pltpu.SemaphoreType.DMA,
            pltpu.SemaphoreType.DMA,
        ],
        compiler_params=pltpu.CompilerParams(use_tc_tiling_on_sc=False),
    )
    def gather_kernel(table_hbm, idx_hbm, out_hbm, shared_t, idx_v, stg,
                      sem_t, sem):
        sid = lax.axis_index("s")
        wid = sid * nc + lax.axis_index("c")
        base = wid * bw
        pltpu.async_copy(table_hbm.at[pl.ds(sid, 1), :],
                         shared_t.at[pl.ds(sid, 1), :], sem_t)
        pltpu.sync_copy(idx_hbm.at[pl.ds(base, bw)], idx_v)
        pltpu.make_async_copy(table_hbm.at[pl.ds(sid, 1), :],
                              shared_t.at[pl.ds(sid, 1), :], sem_t).wait()
        plsc.subcore_barrier()

        def fire(g):
            v = idx_v[pl.ds(g * 16, 16)]
            for j in range(16):
                pltpu.async_copy(
                    shared_t.at[:, pl.ds(v[j], 1)],
                    stg.at[:, pl.ds(g * 16 + j, 1)], sem)

        def drain(g):
            for j in range(16):
                pltpu.make_async_copy(
                    table_hbm.at[:, pl.ds(0, 1)],
                    stg.at[:, pl.ds(g * 16 + j, 1)], sem).wait()

        def body(g, carry):
            fire(g)

            @pl.when(g > 0)
            def _():
                drain(g - 1)

            return carry

        nch = bw // 16
        lax.fori_loop(0, nch, body, 0)
        drain(nch - 1)
        pltpu.sync_copy(stg, out_hbm.at[:, pl.ds(base, bw)])

    return gather_kernel


def _tc_tail_body(ages_ref, feats_ref, gamma_ref, beta_ref, w_ref, b_ref,
                  out_ref):
    a = ages_ref[...]
    n = a.shape[1]
    mean = jnp.sum(a) / n
    centered = a - mean
    var = jnp.sum(centered * centered) / n
    a_hat = centered * lax.rsqrt(var + EPS)
    age_feat = a_hat * gamma_ref[0] + beta_ref[0]
    age_relu = jnp.maximum(age_feat, 0.0)
    feats = jnp.maximum(feats_ref[...], 0.0)
    w = w_ref[...]
    out = lax.dot_general(
        w[:EMB_DIM, :], feats,
        (((0,), (0,)), ((), ())),
        preferred_element_type=jnp.float32,
    )
    out = out + w[EMB_DIM:EMB_DIM + 1, :].reshape(OUT_DIM, 1) * age_relu
    out = out + b_ref[...].reshape(OUT_DIM, 1)
    out_ref[...] = out


def kernel(customer_ids, ages, emb_table, bn_gamma, bn_beta, W, b):
    table_t = emb_table.T
    feats_t = _make_sc_gather(BATCH, EMB_DIM, VOCAB)(
        table_t, customer_ids.astype(jnp.int32))
    out_t = pl.pallas_call(
        _tc_tail_body,
        out_shape=jax.ShapeDtypeStruct((OUT_DIM, BATCH), jnp.float32),
        in_specs=[
            pl.BlockSpec(memory_space=pltpu.VMEM),
            pl.BlockSpec(memory_space=pltpu.VMEM),
            pl.BlockSpec(memory_space=pltpu.SMEM),
            pl.BlockSpec(memory_space=pltpu.SMEM),
            pl.BlockSpec(memory_space=pltpu.VMEM),
            pl.BlockSpec(memory_space=pltpu.VMEM),
        ],
        out_specs=pl.BlockSpec(memory_space=pltpu.VMEM),
    )(ages.reshape(1, BATCH), feats_t, bn_gamma, bn_beta, W,
      b.reshape(1, OUT_DIM))
    return out_t.T
